# baseline (speedup 1.0000x reference)
import functools
import jax
import jax.numpy as jnp
import numpy as np
from jax.experimental import pallas as pl
from jax.experimental.pallas import tpu as pltpu

_NUM_BUS = 50000
_NUM_GEN = 10000
_HID = 32
_HEADS = 2
_C = 32
_HC = _HEADS * _C
_LAYERS = 2
_VM, _VA, _PD, _QD, _QG, _GS, _BS = 0, 1, 2, 3, 4, 5, 6
_ISQC = 1.0 / np.sqrt(_C)


def _ln(x, g, b):
    m = x.mean(-1, keepdims=True)
    v = ((x - m) ** 2).mean(-1, keepdims=True)
    return (x - m) * jax.lax.rsqrt(v + 1e-5) * g + b


def _proj_body(x_ref, w1_ref, b1_ref, w2_ref, b2_ref, g_ref, be_ref, o_ref):
    h = jax.nn.leaky_relu(x_ref[...] @ w1_ref[...] + b1_ref[...])
    h = h @ w2_ref[...] + b2_ref[...]
    o_ref[...] = _ln(h, g_ref[...], be_ref[...])


def _apply_proj(p, x, block=2048):
    n, din = x.shape
    h = p["l1"]["W"].shape[1]
    npad = ((n + block - 1) // block) * block
    xp = jnp.pad(x, ((0, npad - n), (0, 0)))
    out = pl.pallas_call(
        _proj_body,
        grid=(npad // block,),
        in_specs=[
            pl.BlockSpec((block, din), lambda i: (i, 0)),
            pl.BlockSpec((din, h), lambda i: (0, 0)),
            pl.BlockSpec((h,), lambda i: (0,)),
            pl.BlockSpec((h, h), lambda i: (0, 0)),
            pl.BlockSpec((h,), lambda i: (0,)),
            pl.BlockSpec((h,), lambda i: (0,)),
            pl.BlockSpec((h,), lambda i: (0,)),
        ],
        out_specs=pl.BlockSpec((block, h), lambda i: (i, 0)),
        out_shape=jax.ShapeDtypeStruct((npad, h), jnp.float32),
    )(xp, p["l1"]["W"], p["l1"]["b"], p["l2"]["W"], p["l2"]["b"], p["g"], p["be"])
    return out[:n]


def _tconv(p, x_src, x_dst, src, dst, n_dst, e_emb=None):
    q = (x_dst @ p["q"]["W"] + p["q"]["b"]).reshape(-1, _HEADS, _C)
    k = (x_src @ p["k"]["W"] + p["k"]["b"]).reshape(-1, _HEADS, _C)
    v = (x_src @ p["v"]["W"] + p["v"]["b"]).reshape(-1, _HEADS, _C)
    ke = k[src]
    ve = v[src]
    qe = q[dst]
    if e_emb is not None:
        ee = (e_emb @ p["e"]["W"] + p["e"]["b"]).reshape(-1, _HEADS, _C)
        ke = ke + ee
    logits = (qe * ke).sum(-1) * _ISQC
    ex = jnp.exp(logits)
    msg = ve + ee if e_emb is not None else ve
    num = jax.ops.segment_sum(msg * ex[:, :, None], dst, num_segments=n_dst)
    den = jax.ops.segment_sum(ex, dst, num_segments=n_dst)
    out = (num / (den + 1e-16)[:, :, None]).reshape(n_dst, _HC)
    xr = x_dst @ p["skip"]["W"] + p["skip"]["b"]
    bet = jax.nn.sigmoid(jnp.concatenate([out, xr, out - xr], axis=-1) @ p["beta"])
    return bet * xr + (1.0 - bet) * out


def _mlp(p, x):
    h = x @ p["l1"]["W"] + p["l1"]["b"]
    h = jax.nn.leaky_relu(_ln(h, p["g"], p["be"]))
    return h @ p["l2"]["W"] + p["l2"]["b"]


def kernel(params, x_bus, x_gen, edge_attr_bb, ei_bb, ei_gb, ei_bg, mask_bus, mask_gen):
    h_bus = _apply_proj(params["proj_bus"], x_bus)
    h_gen = _apply_proj(params["proj_gen"], x_gen)
    e_emb = _apply_proj(params["proj_edge"], edge_attr_bb)
    bus_mask = mask_bus[:, : _VA + 1]
    gen_mask = mask_gen[:, :1]
    bus_fixed = x_bus[:, : _VA + 1]
    gen_fixed = x_gen[:, :1]
    for i in range(_LAYERS):
        lp = params["layers"][i]
        out_bus = _tconv(lp["bb"], h_bus, h_bus, ei_bb[0], ei_bb[1], _NUM_BUS, e_emb) \
            + _tconv(lp["gb"], h_gen, h_bus, ei_gb[0], ei_gb[1], _NUM_BUS)
        out_gen = _tconv(lp["bg"], h_bus, h_gen, ei_bg[0], ei_bg[1], _NUM_GEN)
        out_bus = jax.nn.leaky_relu(_ln(out_bus, params["norm_bus"][i]["g"], params["norm_bus"][i]["b"]))
        out_gen = jax.nn.leaky_relu(_ln(out_gen, params["norm_gen"][i]["g"], params["norm_gen"][i]["b"]))
        h_bus = h_bus + out_bus if out_bus.shape == h_bus.shape else out_bus
        h_gen = h_gen + out_gen if out_gen.shape == h_gen.shape else out_gen
    bus_temp = _mlp(params["mlp_bus"], h_bus)
    gen_temp = _mlp(params["mlp_gen"], h_gen)
    bus_temp = jnp.where(bus_mask, bus_temp, bus_fixed)
    gen_temp = jnp.where(gen_mask, gen_temp, gen_fixed)

    f, t = ei_bb[0], ei_bb[1]
    Vm = bus_temp[:, _VM]
    Va = bus_temp[:, _VA]
    cr = Vm * jnp.cos(Va)
    ci = Vm * jnp.sin(Va)
    Vf_r = cr[f]
    Vf_i = ci[f]
    Vt_r = cr[t]
    Vt_i = ci[t]
    Yff_r = edge_attr_bb[:, 0]
    Yff_i = edge_attr_bb[:, 1]
    Yft_r = edge_attr_bb[:, 2]
    Yft_i = edge_attr_bb[:, 3]
    Ir = Yff_r * Vf_r - Yff_i * Vf_i + Yft_r * Vt_r - Yft_i * Vt_i
    Ii = Yff_r * Vf_i + Yff_i * Vf_r + Yft_r * Vt_i + Yft_i * Vt_r
    Pft = Vf_r * Ir + Vf_i * Ii
    Qft = Vf_i * Ir - Vf_r * Ii
    P_in = jax.ops.segment_sum(Pft, f, num_segments=_NUM_BUS)
    Q_in = jax.ops.segment_sum(Qft, f, num_segments=_NUM_BUS)
    agg_bus = jax.ops.segment_sum(gen_temp[:, 0], ei_gb[1], num_segments=_NUM_BUS)
    p_sh = -x_bus[:, _GS] * Vm ** 2
    q_sh = x_bus[:, _BS] * Vm ** 2
    residual_P = agg_bus - x_bus[:, _PD] + p_sh - P_in
    residual_Q = x_bus[:, _QG] - x_bus[:, _QD] + q_sh - Q_in
    return (bus_temp, gen_temp, residual_P, residual_Q)


# speedup vs baseline: 7.0508x; 7.0508x over previous
import functools
import jax
import jax.numpy as jnp
import numpy as np
from jax import lax
from jax.experimental import pallas as pl
from jax.experimental.pallas import tpu as pltpu
from jax.experimental.pallas import tpu_sc as plsc

_NUM_BUS = 50000
_NUM_GEN = 10000
_HID = 32
_HEADS = 2
_C = 32
_HC = _HEADS * _C
_LAYERS = 2
_VM, _VA, _PD, _QD, _QG, _GS, _BS = 0, 1, 2, 3, 4, 5, 6
_ISQC = 1.0 / np.sqrt(_C)
_CH = 80
_PW = 40


def _ln(x, g, b):
    m = x.mean(-1, keepdims=True)
    v = ((x - m) ** 2).mean(-1, keepdims=True)
    return (x - m) * jax.lax.rsqrt(v + 1e-5) * g + b


def _proj_body(x_ref, w1_ref, b1_ref, w2_ref, b2_ref, g_ref, be_ref, o_ref):
    h = jax.nn.leaky_relu(x_ref[...] @ w1_ref[...] + b1_ref[...])
    h = h @ w2_ref[...] + b2_ref[...]
    o_ref[...] = _ln(h, g_ref[...], be_ref[...])


def _apply_proj(p, x, block=2048):
    n, din = x.shape
    h = p["l1"]["W"].shape[1]
    npad = ((n + block - 1) // block) * block
    xp = jnp.pad(x, ((0, npad - n), (0, 0)))
    out = pl.pallas_call(
        _proj_body,
        grid=(npad // block,),
        in_specs=[
            pl.BlockSpec((block, din), lambda i: (i, 0)),
            pl.BlockSpec((din, h), lambda i: (0, 0)),
            pl.BlockSpec((h,), lambda i: (0,)),
            pl.BlockSpec((h, h), lambda i: (0, 0)),
            pl.BlockSpec((h,), lambda i: (0,)),
            pl.BlockSpec((h,), lambda i: (0,)),
            pl.BlockSpec((h,), lambda i: (0,)),
        ],
        out_specs=pl.BlockSpec((block, h), lambda i: (i, 0)),
        out_shape=jax.ShapeDtypeStruct((npad, h), jnp.float32),
    )(xp, p["l1"]["W"], p["l1"]["b"], p["l2"]["W"], p["l2"]["b"], p["g"], p["be"])
    return out[:n]


def _mlp(p, x):
    h = x @ p["l1"]["W"] + p["l1"]["b"]
    h = jax.nn.leaky_relu(_ln(h, p["g"], p["be"]))
    return h @ p["l2"]["W"] + p["l2"]["b"]


def _heads_major(x):
    n = x.shape[0]
    return x.reshape(n, _HEADS, _C).transpose(1, 0, 2).reshape(_HEADS * n, _C)


@functools.partial(jax.jit, static_argnames=("n_src", "n_dst", "n_edge", "use_ee"))
def _conv_sc_call(qT, kT, vT, eeT, src, dst, zrows, *, n_src, n_dst, n_edge, use_ee):
    assert n_edge % _CH == 0 and n_dst % 80 == 0
    n_dp = ((n_dst + 127) // 128) * 128
    n_dchunks = n_dp // 128
    djmax = (n_dchunks + 15) // 16
    n_chunks = n_edge // _CH
    jmax = (n_chunks + 15) // 16
    ngrp = _CH // 16
    mesh = plsc.VectorSubcoreMesh(core_axis_name="c", subcore_axis_name="s")

    @functools.partial(
        pl.kernel,
        out_type=(jax.ShapeDtypeStruct((_HEADS, n_dst, _C), jnp.float32),
                  jax.ShapeDtypeStruct((_HEADS * n_dp,), jnp.float32)),
        mesh=mesh,
        compiler_params=pltpu.CompilerParams(use_tc_tiling_on_sc=False),
        scratch_types=[
            pltpu.VMEM((_CH,), jnp.int32),
            pltpu.VMEM((_CH,), jnp.int32),
            pltpu.VMEM((_CH,), jnp.int32),
            pltpu.VMEM((_CH,), jnp.int32),
            pltpu.VMEM((_CH, _C), jnp.float32),
            pltpu.VMEM((_CH, _C), jnp.float32),
            pltpu.VMEM((_CH, _C), jnp.float32),
            pltpu.VMEM((_CH, _C), jnp.float32),
            pltpu.VMEM((_CH, _C), jnp.float32),
            pltpu.VMEM((_CH,), jnp.float32),
            pltpu.VMEM_SHARED((n_dst, _C), jnp.float32),
            pltpu.VMEM_SHARED((n_dp,), jnp.float32),
            pltpu.SemaphoreType.DMA,
        ],
    )
    def conv(qT_r, kT_r, vT_r, eeT_r, src_r, dst_r, z_r, zd_r, out_r, outden_r,
             srcv, srch, dstv, dsth, qb, kb, vb, eb, pay, exb, acc, dacc, sem):
        c = lax.axis_index("c")
        t = lax.axis_index("s")
        it16 = lax.iota(jnp.int32, 16)
        z16 = jnp.zeros((16,), jnp.float32)

        n_rchunks = n_dst // 80
        rjmax = (n_rchunks + 15) // 16
        def zero80(j, _):
            rcid = j * 16 + t
            @pl.when(rcid < n_rchunks)
            def _():
                pltpu.sync_copy(z_r.at[pl.ds(0, 80)],
                                acc.at[pl.ds(rcid * 80, 80)])
            return 0
        lax.fori_loop(0, rjmax, zero80, 0)
        def zeroden(j, _):
            dcid = j * 16 + t
            @pl.when(dcid < n_dchunks)
            def _():
                pltpu.sync_copy(zd_r, dacc.at[pl.ds(dcid * 128, 128)])
            return 0
        lax.fori_loop(0, djmax, zeroden, 0)
        plsc.subcore_barrier()

        cq = c * n_dst
        ck = c * n_src
        ce = c * n_edge
        h0 = pl.ds(0, 16)
        h1 = pl.ds(16, 16)

        def chunk(j, _):
            cid = j * 16 + t
            @pl.when(cid < n_chunks)
            def _():
                base = cid * _CH
                pltpu.sync_copy(src_r.at[pl.ds(base, _CH)], srcv)
                pltpu.sync_copy(dst_r.at[pl.ds(base, _CH)], dstv)
                for g in range(ngrp):
                    sl = pl.ds(g * 16, 16)
                    srch[sl] = srcv[sl] + ck
                    dsth[sl] = dstv[sl] + cq
                cps = [
                    pltpu.async_copy(kT_r.at[srch], kb, sem),
                    pltpu.async_copy(vT_r.at[srch], vb, sem),
                    pltpu.async_copy(qT_r.at[dsth], qb, sem),
                ]
                if use_ee:
                    cps.append(pltpu.async_copy(
                        eeT_r.at[pl.ds(ce + base, _CH)], eb, sem))
                for cp in cps:
                    cp.wait()
                for g in range(ngrp):
                    sl = pl.ds(g * 16, 16)
                    lgv = z16
                    for l in range(16):
                        e = g * 16 + l
                        if use_ee:
                            p = qb[e, h0] * (kb[e, h0] + eb[e, h0]) \
                                + qb[e, h1] * (kb[e, h1] + eb[e, h1])
                        else:
                            p = qb[e, h0] * kb[e, h0] + qb[e, h1] * kb[e, h1]
                        for sh in (8, 4, 2, 1):
                            p = p + p.at[it16 ^ sh].get(
                                mode="promise_in_bounds")
                        lgv = jnp.where(it16 == l, p * _ISQC, lgv)
                    exv = jnp.exp(lgv)
                    exb[sl] = exv
                    for l in range(16):
                        e = g * 16 + l
                        exs = exv[l]
                        if use_ee:
                            pay[e, h0] = exs * (vb[e, h0] + eb[e, h0])
                            pay[e, h1] = exs * (vb[e, h1] + eb[e, h1])
                        else:
                            pay[e, h0] = exs * vb[e, h0]
                            pay[e, h1] = exs * vb[e, h1]
                pltpu.sync_copy(pay, acc.at[dstv], add=True)
                pltpu.sync_copy(exb, dacc.at[dstv], add=True)
            return 0

        lax.fori_loop(0, jmax, chunk, 0)
        plsc.subcore_barrier()
        def flush80(j, _):
            rcid = j * 16 + t
            @pl.when(rcid < n_rchunks)
            def _():
                pltpu.sync_copy(acc.at[pl.ds(rcid * 80, 80)],
                                out_r.at[c, pl.ds(rcid * 80, 80)])
            return 0
        lax.fori_loop(0, rjmax, flush80, 0)
        def flushden(j, _):
            dcid = j * 16 + t
            @pl.when(dcid < n_dchunks)
            def _():
                pltpu.sync_copy(dacc.at[pl.ds(dcid * 128, 128)],
                                outden_r.at[pl.ds(c * n_dp + dcid * 128, 128)])
            return 0
        lax.fori_loop(0, djmax, flushden, 0)

    return conv(qT, kT, vT, eeT, src, dst, zrows, jnp.zeros((128,), jnp.float32))


def _tconv_sc(p, x_src, x_dst, src, dst, n_dst, zrows, e_emb_T=None):
    n_src = x_src.shape[0]
    n_edge = src.shape[0]
    qT = _heads_major(x_dst @ p["q"]["W"] + p["q"]["b"])
    kT = _heads_major(x_src @ p["k"]["W"] + p["k"]["b"])
    vT = _heads_major(x_src @ p["v"]["W"] + p["v"]["b"])
    if e_emb_T is None:
        eeT = jnp.zeros((2, _C), jnp.float32)
    else:
        eeT = e_emb_T
    num, den_t = _conv_sc_call(qT, kT, vT, eeT, src, dst, zrows,
                               n_src=n_src, n_dst=n_dst, n_edge=n_edge,
                               use_ee=e_emb_T is not None)
    n_dp = ((n_dst + 127) // 128) * 128
    den = den_t.reshape(2, n_dp)[:, :n_dst]
    out = (num / (den + 1e-16)[:, :, None]).transpose(1, 0, 2).reshape(n_dst, _HC)
    xr = x_dst @ p["skip"]["W"] + p["skip"]["b"]
    bet = jax.nn.sigmoid(jnp.concatenate([out, xr, out - xr], axis=-1) @ p["beta"])
    return bet * xr + (1.0 - bet) * out


def kernel(params, x_bus, x_gen, edge_attr_bb, ei_bb, ei_gb, ei_bg, mask_bus, mask_gen):
    h_bus = _apply_proj(params["proj_bus"], x_bus)
    h_gen = _apply_proj(params["proj_gen"], x_gen)
    e_emb = _apply_proj(params["proj_edge"], edge_attr_bb)
    zrows = jnp.zeros((128, _C), jnp.float32)
    src_bb, dst_bb = ei_bb[0], ei_bb[1]
    src_gb, dst_gb = ei_gb[0], ei_gb[1]
    src_bg, dst_bg = ei_bg[0], ei_bg[1]
    bus_mask = mask_bus[:, : _VA + 1]
    gen_mask = mask_gen[:, :1]
    bus_fixed = x_bus[:, : _VA + 1]
    gen_fixed = x_gen[:, :1]
    for i in range(_LAYERS):
        lp = params["layers"][i]
        eeT = _heads_major(e_emb @ lp["bb"]["e"]["W"] + lp["bb"]["e"]["b"])
        out_bus = _tconv_sc(lp["bb"], h_bus, h_bus, src_bb, dst_bb, _NUM_BUS, zrows, eeT) \
            + _tconv_sc(lp["gb"], h_gen, h_bus, src_gb, dst_gb, _NUM_BUS, zrows)
        out_gen = _tconv_sc(lp["bg"], h_bus, h_gen, src_bg, dst_bg, _NUM_GEN, zrows)
        out_bus = jax.nn.leaky_relu(_ln(out_bus, params["norm_bus"][i]["g"], params["norm_bus"][i]["b"]))
        out_gen = jax.nn.leaky_relu(_ln(out_gen, params["norm_gen"][i]["g"], params["norm_gen"][i]["b"]))
        h_bus = h_bus + out_bus if out_bus.shape == h_bus.shape else out_bus
        h_gen = h_gen + out_gen if out_gen.shape == h_gen.shape else out_gen
    bus_temp = _mlp(params["mlp_bus"], h_bus)
    gen_temp = _mlp(params["mlp_gen"], h_gen)
    bus_temp = jnp.where(bus_mask, bus_temp, bus_fixed)
    gen_temp = jnp.where(gen_mask, gen_temp, gen_fixed)

    f, t = src_bb, dst_bb
    Vm = bus_temp[:, _VM]
    Va = bus_temp[:, _VA]
    cr = Vm * jnp.cos(Va)
    ci = Vm * jnp.sin(Va)
    Vf_r = cr[f]
    Vf_i = ci[f]
    Vt_r = cr[t]
    Vt_i = ci[t]
    Yff_r = edge_attr_bb[:, 0]
    Yff_i = edge_attr_bb[:, 1]
    Yft_r = edge_attr_bb[:, 2]
    Yft_i = edge_attr_bb[:, 3]
    Ir = Yff_r * Vf_r - Yff_i * Vf_i + Yft_r * Vt_r - Yft_i * Vt_i
    Ii = Yff_r * Vf_i + Yff_i * Vf_r + Yft_r * Vt_i + Yft_i * Vt_r
    Pft = Vf_r * Ir + Vf_i * Ii
    Qft = Vf_i * Ir - Vf_r * Ii
    P_in = jax.ops.segment_sum(Pft, f, num_segments=_NUM_BUS)
    Q_in = jax.ops.segment_sum(Qft, f, num_segments=_NUM_BUS)
    agg_bus = jax.ops.segment_sum(gen_temp[:, 0], dst_gb, num_segments=_NUM_BUS)
    p_sh = -x_bus[:, _GS] * Vm ** 2
    q_sh = x_bus[:, _BS] * Vm ** 2
    residual_P = agg_bus - x_bus[:, _PD] + p_sh - P_in
    residual_Q = x_bus[:, _QG] - x_bus[:, _QD] + q_sh - Q_in
    return (bus_temp, gen_temp, residual_P, residual_Q)


# speedup vs baseline: 23.1809x; 3.2877x over previous
import functools
import jax
import jax.numpy as jnp
import numpy as np
from jax import lax
from jax.experimental import pallas as pl
from jax.experimental.pallas import tpu as pltpu
from jax.experimental.pallas import tpu_sc as plsc

_NUM_BUS = 50000
_NUM_GEN = 10000
_HID = 32
_HEADS = 2
_C = 32
_HC = _HEADS * _C
_LAYERS = 2
_VM, _VA, _PD, _QD, _QG, _GS, _BS = 0, 1, 2, 3, 4, 5, 6
_ISQC = 1.0 / np.sqrt(_C)
_CH = 80
_PW = 40


def _ln(x, g, b):
    m = x.mean(-1, keepdims=True)
    v = ((x - m) ** 2).mean(-1, keepdims=True)
    return (x - m) * jax.lax.rsqrt(v + 1e-5) * g + b


def _proj_body(x_ref, w1_ref, b1_ref, w2_ref, b2_ref, g_ref, be_ref, o_ref):
    h = jax.nn.leaky_relu(x_ref[...] @ w1_ref[...] + b1_ref[...])
    h = h @ w2_ref[...] + b2_ref[...]
    o_ref[...] = _ln(h, g_ref[...], be_ref[...])


def _apply_proj(p, x, block=2048):
    n, din = x.shape
    h = p["l1"]["W"].shape[1]
    npad = ((n + block - 1) // block) * block
    xp = jnp.pad(x, ((0, npad - n), (0, 0)))
    out = pl.pallas_call(
        _proj_body,
        grid=(npad // block,),
        in_specs=[
            pl.BlockSpec((block, din), lambda i: (i, 0)),
            pl.BlockSpec((din, h), lambda i: (0, 0)),
            pl.BlockSpec((h,), lambda i: (0,)),
            pl.BlockSpec((h, h), lambda i: (0, 0)),
            pl.BlockSpec((h,), lambda i: (0,)),
            pl.BlockSpec((h,), lambda i: (0,)),
            pl.BlockSpec((h,), lambda i: (0,)),
        ],
        out_specs=pl.BlockSpec((block, h), lambda i: (i, 0)),
        out_shape=jax.ShapeDtypeStruct((npad, h), jnp.float32),
    )(xp, p["l1"]["W"], p["l1"]["b"], p["l2"]["W"], p["l2"]["b"], p["g"], p["be"])
    return out[:n]


def _mlp(p, x):
    h = x @ p["l1"]["W"] + p["l1"]["b"]
    h = jax.nn.leaky_relu(_ln(h, p["g"], p["be"]))
    return h @ p["l2"]["W"] + p["l2"]["b"]


def _heads_major(x):
    n = x.shape[0]
    return x.reshape(n, _HEADS, _C).transpose(1, 0, 2).reshape(_HEADS * n, _C)


@functools.partial(jax.jit, static_argnames=("n_src", "n_dst", "n_edge", "use_ee"))
def _conv_sc_call(qT, kT, vT, eeT, src, dst, zrows, *, n_src, n_dst, n_edge, use_ee):
    assert n_edge % _CH == 0 and n_dst % 80 == 0
    n_dp = ((n_dst + 127) // 128) * 128
    n_dchunks = n_dp // 128
    djmax = (n_dchunks + 15) // 16
    n_chunks = n_edge // _CH
    jmax = (n_chunks + 15) // 16
    ngrp = _CH // 16
    mesh = plsc.VectorSubcoreMesh(core_axis_name="c", subcore_axis_name="s")

    @functools.partial(
        pl.kernel,
        out_type=(jax.ShapeDtypeStruct((_HEADS, n_dst, _C), jnp.float32),
                  jax.ShapeDtypeStruct((_HEADS * n_dp,), jnp.float32)),
        mesh=mesh,
        compiler_params=pltpu.CompilerParams(use_tc_tiling_on_sc=False),
        scratch_types=[
            pltpu.VMEM((_CH,), jnp.int32),
            pltpu.VMEM((_CH,), jnp.int32),
            pltpu.VMEM((_CH,), jnp.int32),
            pltpu.VMEM((_CH,), jnp.int32),
            pltpu.VMEM((_CH, _C), jnp.float32),
            pltpu.VMEM((_CH, _C), jnp.float32),
            pltpu.VMEM((_CH, _C), jnp.float32),
            pltpu.VMEM((_CH, _C), jnp.float32),
            pltpu.VMEM((_CH, _C), jnp.float32),
            pltpu.VMEM((_CH,), jnp.float32),
            pltpu.VMEM_SHARED((n_dst, _C), jnp.float32),
            pltpu.VMEM_SHARED((n_dp,), jnp.float32),
            pltpu.SemaphoreType.DMA,
        ],
    )
    def conv(qT_r, kT_r, vT_r, eeT_r, src_r, dst_r, z_r, zd_r, out_r, outden_r,
             srcv, srch, dstv, dsth, qb, kb, vb, eb, pay, exb, acc, dacc, sem):
        c = lax.axis_index("c")
        t = lax.axis_index("s")
        it16 = lax.iota(jnp.int32, 16)
        z16 = jnp.zeros((16,), jnp.float32)

        n_rchunks = n_dst // 80
        rjmax = (n_rchunks + 15) // 16
        def zero80(j, _):
            rcid = j * 16 + t
            @pl.when(rcid < n_rchunks)
            def _():
                pltpu.sync_copy(z_r.at[pl.ds(0, 80)],
                                acc.at[pl.ds(rcid * 80, 80)])
            return 0
        lax.fori_loop(0, rjmax, zero80, 0)
        def zeroden(j, _):
            dcid = j * 16 + t
            @pl.when(dcid < n_dchunks)
            def _():
                pltpu.sync_copy(zd_r, dacc.at[pl.ds(dcid * 128, 128)])
            return 0
        lax.fori_loop(0, djmax, zeroden, 0)
        plsc.subcore_barrier()

        cq = c * n_dst
        ck = c * n_src
        ce = c * n_edge
        h0 = pl.ds(0, 16)
        h1 = pl.ds(16, 16)

        def chunk(j, _):
            cid = j * 16 + t
            @pl.when(cid < n_chunks)
            def _():
                base = cid * _CH
                pltpu.sync_copy(src_r.at[pl.ds(base, _CH)], srcv)
                pltpu.sync_copy(dst_r.at[pl.ds(base, _CH)], dstv)
                for g in range(ngrp):
                    sl = pl.ds(g * 16, 16)
                    srch[sl] = srcv[sl] + ck
                    dsth[sl] = dstv[sl] + cq
                cps = [
                    pltpu.async_copy(kT_r.at[srch], kb, sem),
                    pltpu.async_copy(vT_r.at[srch], vb, sem),
                    pltpu.async_copy(qT_r.at[dsth], qb, sem),
                ]
                if use_ee:
                    cps.append(pltpu.async_copy(
                        eeT_r.at[pl.ds(ce + base, _CH)], eb, sem))
                for cp in cps:
                    cp.wait()
                for g in range(ngrp):
                    sl = pl.ds(g * 16, 16)
                    lgv = z16
                    for l in range(16):
                        e = g * 16 + l
                        if use_ee:
                            p = qb[e, h0] * (kb[e, h0] + eb[e, h0]) \
                                + qb[e, h1] * (kb[e, h1] + eb[e, h1])
                        else:
                            p = qb[e, h0] * kb[e, h0] + qb[e, h1] * kb[e, h1]
                        for sh in (8, 4, 2, 1):
                            p = p + p.at[it16 ^ sh].get(
                                mode="promise_in_bounds")
                        lgv = jnp.where(it16 == l, p * _ISQC, lgv)
                    exv = jnp.exp(lgv)
                    exb[sl] = exv
                    for l in range(16):
                        e = g * 16 + l
                        exs = exv[l]
                        if use_ee:
                            pay[e, h0] = exs * (vb[e, h0] + eb[e, h0])
                            pay[e, h1] = exs * (vb[e, h1] + eb[e, h1])
                        else:
                            pay[e, h0] = exs * vb[e, h0]
                            pay[e, h1] = exs * vb[e, h1]
                pltpu.sync_copy(pay, acc.at[dstv], add=True)
                pltpu.sync_copy(exb, dacc.at[dstv], add=True)
            return 0

        lax.fori_loop(0, jmax, chunk, 0)
        plsc.subcore_barrier()
        def flush80(j, _):
            rcid = j * 16 + t
            @pl.when(rcid < n_rchunks)
            def _():
                pltpu.sync_copy(acc.at[pl.ds(rcid * 80, 80)],
                                out_r.at[c, pl.ds(rcid * 80, 80)])
            return 0
        lax.fori_loop(0, rjmax, flush80, 0)
        def flushden(j, _):
            dcid = j * 16 + t
            @pl.when(dcid < n_dchunks)
            def _():
                pltpu.sync_copy(dacc.at[pl.ds(dcid * 128, 128)],
                                outden_r.at[pl.ds(c * n_dp + dcid * 128, 128)])
            return 0
        lax.fori_loop(0, djmax, flushden, 0)

    return conv(qT, kT, vT, eeT, src, dst, zrows, jnp.zeros((128,), jnp.float32))


def _tconv_sc(p, x_src, x_dst, src, dst, n_dst, zrows, e_emb_T=None):
    n_src = x_src.shape[0]
    n_edge = src.shape[0]
    qT = _heads_major(x_dst @ p["q"]["W"] + p["q"]["b"])
    kT = _heads_major(x_src @ p["k"]["W"] + p["k"]["b"])
    vT = _heads_major(x_src @ p["v"]["W"] + p["v"]["b"])
    if e_emb_T is None:
        eeT = jnp.zeros((2, _C), jnp.float32)
    else:
        eeT = e_emb_T
    num, den_t = _conv_sc_call(qT, kT, vT, eeT, src, dst, zrows,
                               n_src=n_src, n_dst=n_dst, n_edge=n_edge,
                               use_ee=e_emb_T is not None)
    n_dp = ((n_dst + 127) // 128) * 128
    den = den_t.reshape(2, n_dp)[:, :n_dst]
    out = (num / (den + 1e-16)[:, :, None]).transpose(1, 0, 2).reshape(n_dst, _HC)
    xr = x_dst @ p["skip"]["W"] + p["skip"]["b"]
    bet = jax.nn.sigmoid(jnp.concatenate([out, xr, out - xr], axis=-1) @ p["beta"])
    return bet * xr + (1.0 - bet) * out


@jax.jit
def _phys_sc_call(cr, ci, yffr, yffi, yftr, yfti, f, t):
    n_edge = f.shape[0]
    assert n_edge % _CH == 0
    n_bp = ((_NUM_BUS + 127) // 128) * 128
    n_chunks = n_edge // _CH
    jmax = (n_chunks + 31) // 32
    n_dchunks = n_bp // 128
    djmax = (n_dchunks + 15) // 16
    ngrp = _CH // 16
    mesh = plsc.VectorSubcoreMesh(core_axis_name="c", subcore_axis_name="s")

    @functools.partial(
        pl.kernel,
        out_type=(jax.ShapeDtypeStruct((2 * n_bp,), jnp.float32),
                  jax.ShapeDtypeStruct((2 * n_bp,), jnp.float32)),
        mesh=mesh,
        compiler_params=pltpu.CompilerParams(use_tc_tiling_on_sc=False),
        scratch_types=[
            pltpu.VMEM((_CH,), jnp.int32),
            pltpu.VMEM((_CH,), jnp.int32),
            pltpu.VMEM((_CH,), jnp.float32),
            pltpu.VMEM((_CH,), jnp.float32),
            pltpu.VMEM((_CH,), jnp.float32),
            pltpu.VMEM((_CH,), jnp.float32),
            pltpu.VMEM((_CH,), jnp.float32),
            pltpu.VMEM((_CH,), jnp.float32),
            pltpu.VMEM((_CH,), jnp.float32),
            pltpu.VMEM((_CH,), jnp.float32),
            pltpu.VMEM((_CH,), jnp.float32),
            pltpu.VMEM((_CH,), jnp.float32),
            pltpu.VMEM_SHARED((n_bp,), jnp.float32),
            pltpu.VMEM_SHARED((n_bp,), jnp.float32),
            pltpu.SemaphoreType.DMA,
        ],
    )
    def phys(cr_r, ci_r, y0_r, y1_r, y2_r, y3_r, f_r, t_r, zd_r, outp_r, outq_r,
             fidx, tidx, vfr, vfi, vtr, vti, y0, y1, y2, y3, pp, qq,
             accP, accQ, sem):
        c = lax.axis_index("c")
        tt = lax.axis_index("s")
        w = tt + 16 * c

        def zeroden(j, _):
            dcid = j * 16 + tt
            @pl.when(dcid < n_dchunks)
            def _():
                pltpu.sync_copy(zd_r, accP.at[pl.ds(dcid * 128, 128)])
                pltpu.sync_copy(zd_r, accQ.at[pl.ds(dcid * 128, 128)])
            return 0
        lax.fori_loop(0, djmax, zeroden, 0)
        plsc.subcore_barrier()

        def chunk(j, _):
            cid = j * 32 + w
            @pl.when(cid < n_chunks)
            def _():
                base = cid * _CH
                pltpu.sync_copy(f_r.at[pl.ds(base, _CH)], fidx)
                pltpu.sync_copy(t_r.at[pl.ds(base, _CH)], tidx)
                cps = [
                    pltpu.async_copy(cr_r.at[fidx], vfr, sem),
                    pltpu.async_copy(ci_r.at[fidx], vfi, sem),
                    pltpu.async_copy(cr_r.at[tidx], vtr, sem),
                    pltpu.async_copy(ci_r.at[tidx], vti, sem),
                    pltpu.async_copy(y0_r.at[pl.ds(base, _CH)], y0, sem),
                    pltpu.async_copy(y1_r.at[pl.ds(base, _CH)], y1, sem),
                    pltpu.async_copy(y2_r.at[pl.ds(base, _CH)], y2, sem),
                    pltpu.async_copy(y3_r.at[pl.ds(base, _CH)], y3, sem),
                ]
                for cp in cps:
                    cp.wait()
                for g in range(ngrp):
                    sl = pl.ds(g * 16, 16)
                    a = vfr[sl]
                    b = vfi[sl]
                    d = vtr[sl]
                    e2 = vti[sl]
                    g0 = y0[sl]
                    g1 = y1[sl]
                    g2 = y2[sl]
                    g3 = y3[sl]
                    ir = g0 * a - g1 * b + g2 * d - g3 * e2
                    ii = g0 * b + g1 * a + g2 * e2 + g3 * d
                    pp[sl] = a * ir + b * ii
                    qq[sl] = b * ir - a * ii
                pltpu.sync_copy(pp, accP.at[fidx], add=True)
                pltpu.sync_copy(qq, accQ.at[fidx], add=True)
            return 0
        lax.fori_loop(0, jmax, chunk, 0)
        plsc.subcore_barrier()

        def flushden(j, _):
            dcid = j * 16 + tt
            @pl.when(dcid < n_dchunks)
            def _():
                pltpu.sync_copy(accP.at[pl.ds(dcid * 128, 128)],
                                outp_r.at[pl.ds(c * n_bp + dcid * 128, 128)])
                pltpu.sync_copy(accQ.at[pl.ds(dcid * 128, 128)],
                                outq_r.at[pl.ds(c * n_bp + dcid * 128, 128)])
            return 0
        lax.fori_loop(0, djmax, flushden, 0)

    return phys(cr, ci, yffr, yffi, yftr, yfti, f, t,
                jnp.zeros((128,), jnp.float32))


def kernel(params, x_bus, x_gen, edge_attr_bb, ei_bb, ei_gb, ei_bg, mask_bus, mask_gen):
    h_bus = _apply_proj(params["proj_bus"], x_bus)
    h_gen = _apply_proj(params["proj_gen"], x_gen)
    e_emb = _apply_proj(params["proj_edge"], edge_attr_bb)
    zrows = jnp.zeros((128, _C), jnp.float32)
    src_bb, dst_bb = ei_bb[0], ei_bb[1]
    src_gb, dst_gb = ei_gb[0], ei_gb[1]
    src_bg, dst_bg = ei_bg[0], ei_bg[1]
    bus_mask = mask_bus[:, : _VA + 1]
    gen_mask = mask_gen[:, :1]
    bus_fixed = x_bus[:, : _VA + 1]
    gen_fixed = x_gen[:, :1]
    for i in range(_LAYERS):
        lp = params["layers"][i]
        eeT = _heads_major(e_emb @ lp["bb"]["e"]["W"] + lp["bb"]["e"]["b"])
        out_bus = _tconv_sc(lp["bb"], h_bus, h_bus, src_bb, dst_bb, _NUM_BUS, zrows, eeT) \
            + _tconv_sc(lp["gb"], h_gen, h_bus, src_gb, dst_gb, _NUM_BUS, zrows)
        out_gen = _tconv_sc(lp["bg"], h_bus, h_gen, src_bg, dst_bg, _NUM_GEN, zrows)
        out_bus = jax.nn.leaky_relu(_ln(out_bus, params["norm_bus"][i]["g"], params["norm_bus"][i]["b"]))
        out_gen = jax.nn.leaky_relu(_ln(out_gen, params["norm_gen"][i]["g"], params["norm_gen"][i]["b"]))
        h_bus = h_bus + out_bus if out_bus.shape == h_bus.shape else out_bus
        h_gen = h_gen + out_gen if out_gen.shape == h_gen.shape else out_gen
    bus_temp = _mlp(params["mlp_bus"], h_bus)
    gen_temp = _mlp(params["mlp_gen"], h_gen)
    bus_temp = jnp.where(bus_mask, bus_temp, bus_fixed)
    gen_temp = jnp.where(gen_mask, gen_temp, gen_fixed)

    f, t = src_bb, dst_bb
    Vm = bus_temp[:, _VM]
    Va = bus_temp[:, _VA]
    cr = Vm * jnp.cos(Va)
    ci = Vm * jnp.sin(Va)
    Yff_r = edge_attr_bb[:, 0]
    Yff_i = edge_attr_bb[:, 1]
    Yft_r = edge_attr_bb[:, 2]
    Yft_i = edge_attr_bb[:, 3]
    P_t, Q_t = _phys_sc_call(cr, ci, Yff_r, Yff_i, Yft_r, Yft_i, f, t)
    n_bp = ((_NUM_BUS + 127) // 128) * 128
    P_in = P_t.reshape(2, n_bp)[:, :_NUM_BUS].sum(0)
    Q_in = Q_t.reshape(2, n_bp)[:, :_NUM_BUS].sum(0)
    agg_bus = jax.ops.segment_sum(gen_temp[:, 0], dst_gb, num_segments=_NUM_BUS)
    p_sh = -x_bus[:, _GS] * Vm ** 2
    q_sh = x_bus[:, _BS] * Vm ** 2
    residual_P = agg_bus - x_bus[:, _PD] + p_sh - P_in
    residual_Q = x_bus[:, _QG] - x_bus[:, _QD] + q_sh - Q_in
    return (bus_temp, gen_temp, residual_P, residual_Q)
